# baseline (speedup 1.0000x reference)
import jax
import jax.numpy as jnp
from jax.experimental import pallas as pl
from jax.experimental.pallas import tpu as pltpu

_VAR0 = 0.1
_VAR1 = 0.2
_THRESH = 0.5
_RATIO = 3


def _dotf(a, b):
    return jax.lax.dot_general(a, b, (((1,), (0,)), ((), ())),
                               preferred_element_type=jnp.float32)


def _prep_priors(pri_ref, p_scr):
    pcx = pri_ref[0:1, :]
    pcy = pri_ref[1:2, :]
    pw = pri_ref[2:3, :]
    ph = pri_ref[3:4, :]
    px1 = pcx - pw / 2.0
    py1 = pcy - ph / 2.0
    px2 = pcx + pw / 2.0
    py2 = pcy + ph / 2.0
    p_scr[0:1, :] = px1
    p_scr[1:2, :] = py1
    p_scr[2:3, :] = px2
    p_scr[3:4, :] = py2
    p_scr[4:5, :] = (px2 - px1) * (py2 - py1)
    p_scr[5:6, :] = 1.0 / (_VAR0 * pw)
    p_scr[6:7, :] = 1.0 / (_VAR0 * ph)
    p_scr[7:8, :] = 1.0 / pw
    p_scr[8:9, :] = 1.0 / ph


def _two_images(g, tgt_ref, tgtT_ref, pri_ref, conf_ref, loc_ref, p_scr,
                store):
    n = tgt_ref.shape[1]
    C = conf_ref.shape[1]
    P = pri_ref.shape[1]

    pcx = pri_ref[0:1, :]
    pcy = pri_ref[1:2, :]

    t2 = jnp.concatenate([tgt_ref[0], tgt_ref[1]], axis=0)
    tx1 = t2[:, 0:1]
    ty1 = t2[:, 1:2]
    tx2 = t2[:, 2:3]
    ty2 = t2[:, 3:4]

    px1 = p_scr[0:1, :]
    py1 = p_scr[1:2, :]
    px2 = p_scr[2:3, :]
    py2 = p_scr[3:4, :]
    parea = p_scr[4:5, :]
    tarea = (tx2 - tx1) * (ty2 - ty1)

    ix = jnp.minimum(tx2, px2) - jnp.maximum(tx1, px1)
    iy = jnp.minimum(ty2, py2) - jnp.maximum(ty1, py1)
    inter = jnp.maximum(ix, 0.0) * jnp.maximum(iy, 0.0)
    ov2 = inter / (tarea + parea - inter)

    i_tru = jax.lax.broadcasted_iota(jnp.int32, (n, P), 0)
    i_lane = jax.lax.broadcasted_iota(jnp.int32, (n, P), 1)
    i_cls = jax.lax.broadcasted_iota(jnp.int32, (C, P), 0)
    ones_c = jnp.ones((1, C), jnp.float32)

    def one_image(r):
        ov = jax.lax.slice_in_dim(ov2, r * n, (r + 1) * n, axis=0)

        bto = jnp.max(ov, axis=0, keepdims=True)
        bti = jnp.min(jnp.where(ov == bto, i_tru, n), axis=0, keepdims=True)

        bpv = jnp.max(ov, axis=1, keepdims=True)
        bpi = jnp.min(jnp.where(ov == bpv, i_lane, P), axis=1, keepdims=True)
        eqm = bpi == i_lane
        fj = jnp.max(jnp.where(eqm, i_tru, -1), axis=0, keepdims=True)
        forced = fj >= 0
        bti = jnp.where(forced, fj, bti)
        bto = jnp.where(forced, 2.0, bto)
        pos = bto >= _THRESH

        ohf = (bti == i_tru).astype(jnp.float32)
        mt = _dotf(tgtT_ref[r], ohf)
        mx1 = mt[0:1]
        my1 = mt[1:2]
        mx2 = mt[2:3]
        my2 = mt[3:4]
        conf_t = jnp.where(pos, (mt[4:5] + 1.5).astype(jnp.int32), 0)

        g0 = ((mx1 + mx2) / 2.0 - pcx) * p_scr[5:6, :]
        g1 = ((my1 + my2) / 2.0 - pcy) * p_scr[6:7, :]
        g2 = jnp.log((mx2 - mx1) * p_scr[7:8, :]) / _VAR1
        g3 = jnp.log((my2 - my1) * p_scr[8:9, :]) / _VAR1

        def sl1(d):
            a = jnp.abs(d)
            return jnp.where(a < 1.0, 0.5 * d * d, a - 0.5)

        loc = loc_ref[r]
        lrow = (sl1(loc[0:1] - g0) + sl1(loc[1:2] - g1) +
                sl1(loc[2:3] - g2) + sl1(loc[3:4] - g3))
        ll = jnp.sum(jnp.where(pos, lrow, 0.0), axis=1, keepdims=True)

        cf = conf_ref[r]
        cmax = jnp.max(cf, axis=0, keepdims=True)
        e = jnp.exp(cf - cmax)
        lse = jnp.log(_dotf(ones_c, e)) + cmax
        gat = _dotf(ones_c, jnp.where(conf_t == i_cls, cf, 0.0))
        ce = lse - gat

        posf = pos.astype(jnp.float32)
        npos = jnp.sum(posf, axis=1, keepdims=True)
        pce = jnp.sum(jnp.where(pos, ce, 0.0), axis=1, keepdims=True)

        store(2 * g + r, jnp.where(pos, 0.0, ce), npos, ll, pce)

    one_image(0)
    one_image(1)


def _phase_kernel(K, final, *refs):
    ins = refs[:5]
    rest = refs[5:]
    if K:
        vc, npc, llc, pcc = rest[:4]
        rest = rest[4:]
    g = pl.program_id(0)
    G = pl.num_programs(0)

    if not final:
        v_out, np_out, ll_out, pc_out, p_scr = rest

        @pl.when(g == 0)
        def _prep():
            _prep_priors(ins[2], p_scr)
            if K:
                v_out[0:K, :] = vc[:, :]
                np_out[0:K, :] = npc[:, :]
                ll_out[0:K, :] = llc[:, :]
                pc_out[0:K, :] = pcc[:, :]

        def store(row, v, npos, ll, pce):
            v_out[pl.ds(K + row, 1), :] = v
            np_out[pl.ds(K + row, 1), :] = npos
            ll_out[pl.ds(K + row, 1), :] = ll
            pc_out[pl.ds(K + row, 1), :] = pce

        _two_images(g, *ins, p_scr, store)
        return

    out_l_ref, out_c_ref, v_scr, np_scr, ll_scr, pc_scr, p_scr = rest
    P = ins[2].shape[1]
    B = v_scr.shape[0]

    @pl.when(g == 0)
    def _prep():
        _prep_priors(ins[2], p_scr)
        v_scr[0:K, :] = vc[:, :]
        np_scr[0:K, :] = npc[:, :]
        ll_scr[0:K, :] = llc[:, :]
        pc_scr[0:K, :] = pcc[:, :]

    def store(row, v, npos, ll, pce):
        v_scr[pl.ds(K + row, 1), :] = v
        np_scr[pl.ds(K + row, 1), :] = npos
        ll_scr[pl.ds(K + row, 1), :] = ll
        pc_scr[pl.ds(K + row, 1), :] = pce

    _two_images(g, *ins, p_scr, store)

    out_l_ref[:, :] = jnp.zeros((1, 1), jnp.float32)
    out_c_ref[:, :] = jnp.zeros((1, 1), jnp.float32)

    @pl.when(g == G - 1)
    def _finalize():
        v = v_scr[:, :]
        vb = jax.lax.bitcast_convert_type(v, jnp.int32)
        npf = np_scr[:, :]
        kf = jnp.minimum(_RATIO * npf, float(P - 1))

        def body(_, lh):
            lo, hi = lh
            mid = lo + (hi - lo + 1) // 2
            cnt = jnp.sum((vb >= mid).astype(jnp.float32), axis=1,
                          keepdims=True)
            ok = cnt >= kf
            return jnp.where(ok, mid, lo), jnp.where(ok, hi, mid - 1)

        lo0 = jnp.zeros((B, 1), jnp.int32)
        hi0 = jnp.full((B, 1), 0x7F7FFFFF, jnp.int32)
        lo, _ = jax.lax.fori_loop(0, 31, body, (lo0, hi0))
        tf = jax.lax.bitcast_convert_type(lo, jnp.float32)
        gt = vb > lo
        sgt = jnp.sum(jnp.where(gt, v, 0.0), axis=1, keepdims=True)
        cgt = jnp.sum(gt.astype(jnp.float32), axis=1, keepdims=True)
        top = sgt + (kf - cgt) * tf

        loss_c = jnp.sum(pc_scr[:, :] + top, axis=0, keepdims=True)
        loss_l = jnp.sum(ll_scr[:, :], axis=0, keepdims=True)
        nn = jnp.sum(np_scr[:, :], axis=0, keepdims=True)
        out_l_ref[:, :] = loss_l / nn
        out_c_ref[:, :] = loss_c / nn


def kernel(loc_data, conf_data, priors, targets):
    import functools
    B, P, C = conf_data.shape
    n = targets.shape[1]
    pri_t = priors.T
    loc_cp = loc_data.transpose(0, 2, 1)
    tgt_t = targets.transpose(0, 2, 1)

    splits = [4, 8, 20] if B == 32 else [B]
    params = pltpu.CompilerParams(dimension_semantics=("arbitrary",))

    def rowspecs(m):
        return [pl.BlockSpec((m, P), lambda b: (0, 0))] + \
               [pl.BlockSpec((m, 1), lambda b: (0, 0))] * 3

    def rowshapes(m):
        return [jax.ShapeDtypeStruct((m, P), jnp.float32)] + \
               [jax.ShapeDtypeStruct((m, 1), jnp.float32)] * 3

    carry = ()
    K = 0
    for i, q in enumerate(splits):
        lo, hi = K, K + q
        final = i == len(splits) - 1
        conf_i = conf_data[lo:hi].transpose(0, 2, 1)
        common_in = [
            pl.BlockSpec((2, n, 5), lambda b: (b, 0, 0)),
            pl.BlockSpec((2, 5, n), lambda b: (b, 0, 0)),
            pl.BlockSpec((4, P), lambda b: (0, 0)),
            pl.BlockSpec((2, C, P), lambda b: (b, 0, 0)),
            pl.BlockSpec((2, 4, P), lambda b: (b, 0, 0)),
        ]
        carry_specs = rowspecs(K) if K else []
        args = (targets[lo:hi], tgt_t[lo:hi], pri_t, conf_i,
                loc_cp[lo:hi]) + carry
        if not final:
            carry = pl.pallas_call(
                functools.partial(_phase_kernel, K, False),
                grid=(q // 2,),
                in_specs=common_in + carry_specs,
                out_specs=rowspecs(hi),
                out_shape=rowshapes(hi),
                scratch_shapes=[pltpu.VMEM((16, P), jnp.float32)],
                compiler_params=params,
            )(*args)
            carry = tuple(carry)
        else:
            out_l, out_c = pl.pallas_call(
                functools.partial(_phase_kernel, K, True),
                grid=(q // 2,),
                in_specs=common_in + carry_specs,
                out_specs=[
                    pl.BlockSpec((1, 1), lambda b: (0, 0)),
                    pl.BlockSpec((1, 1), lambda b: (0, 0)),
                ],
                out_shape=[
                    jax.ShapeDtypeStruct((1, 1), jnp.float32),
                    jax.ShapeDtypeStruct((1, 1), jnp.float32),
                ],
                scratch_shapes=[
                    pltpu.VMEM((B, P), jnp.float32),
                    pltpu.VMEM((B, 1), jnp.float32),
                    pltpu.VMEM((B, 1), jnp.float32),
                    pltpu.VMEM((B, 1), jnp.float32),
                    pltpu.VMEM((16, P), jnp.float32),
                ],
                compiler_params=params,
            )(*args)
        K = hi
    return out_l[0, 0], out_c[0, 0]


# speedup vs baseline: 1.4625x; 1.4625x over previous
import jax
import jax.numpy as jnp
from jax.experimental import pallas as pl
from jax.experimental.pallas import tpu as pltpu

_VAR0 = 0.1
_VAR1 = 0.2
_THRESH = 0.5
_RATIO = 3


def _dotf(a, b):
    return jax.lax.dot_general(a, b, (((1,), (0,)), ((), ())),
                               preferred_element_type=jnp.float32)


def _prep_priors(pri_ref, p_scr):
    pcx = pri_ref[0:1, :]
    pcy = pri_ref[1:2, :]
    pw = pri_ref[2:3, :]
    ph = pri_ref[3:4, :]
    px1 = pcx - pw / 2.0
    py1 = pcy - ph / 2.0
    px2 = pcx + pw / 2.0
    py2 = pcy + ph / 2.0
    p_scr[0:1, :] = px1
    p_scr[1:2, :] = py1
    p_scr[2:3, :] = px2
    p_scr[3:4, :] = py2
    p_scr[4:5, :] = (px2 - px1) * (py2 - py1)
    p_scr[5:6, :] = 1.0 / (_VAR0 * pw)
    p_scr[6:7, :] = 1.0 / (_VAR0 * ph)
    p_scr[7:8, :] = 1.0 / pw
    p_scr[8:9, :] = 1.0 / ph


def _two_images(g, tgt_ref, tgtT_ref, pri_ref, conf_ref, loc_ref, p_scr,
                store):
    n = tgt_ref.shape[1]
    C = conf_ref.shape[1]
    P = pri_ref.shape[1]

    pcx = pri_ref[0:1, :]
    pcy = pri_ref[1:2, :]

    t2 = jnp.concatenate([tgt_ref[0], tgt_ref[1]], axis=0)
    tx1 = t2[:, 0:1]
    ty1 = t2[:, 1:2]
    tx2 = t2[:, 2:3]
    ty2 = t2[:, 3:4]

    px1 = p_scr[0:1, :]
    py1 = p_scr[1:2, :]
    px2 = p_scr[2:3, :]
    py2 = p_scr[3:4, :]
    parea = p_scr[4:5, :]
    tarea = (tx2 - tx1) * (ty2 - ty1)

    ix = jnp.minimum(tx2, px2) - jnp.maximum(tx1, px1)
    iy = jnp.minimum(ty2, py2) - jnp.maximum(ty1, py1)
    inter = jnp.maximum(ix, 0.0) * jnp.maximum(iy, 0.0)
    ov2 = inter / (tarea + parea - inter)

    i_tru = jax.lax.broadcasted_iota(jnp.int32, (n, P), 0)
    i_lane = jax.lax.broadcasted_iota(jnp.int32, (n, P), 1)
    i_cls = jax.lax.broadcasted_iota(jnp.int32, (C, P), 0)
    ones_c = jnp.ones((1, C), jnp.float32)

    def one_image(r):
        ov = jax.lax.slice_in_dim(ov2, r * n, (r + 1) * n, axis=0)

        bto = jnp.max(ov, axis=0, keepdims=True)
        bti = jnp.min(jnp.where(ov == bto, i_tru, n), axis=0, keepdims=True)

        bpv = jnp.max(ov, axis=1, keepdims=True)
        bpi = jnp.min(jnp.where(ov == bpv, i_lane, P), axis=1, keepdims=True)
        eqm = bpi == i_lane
        fj = jnp.max(jnp.where(eqm, i_tru, -1), axis=0, keepdims=True)
        forced = fj >= 0
        bti = jnp.where(forced, fj, bti)
        bto = jnp.where(forced, 2.0, bto)
        pos = bto >= _THRESH

        ohf = (bti == i_tru).astype(jnp.float32)
        mt = _dotf(tgtT_ref[r], ohf)
        mx1 = mt[0:1]
        my1 = mt[1:2]
        mx2 = mt[2:3]
        my2 = mt[3:4]
        conf_t = jnp.where(pos, (mt[4:5] + 1.5).astype(jnp.int32), 0)

        g0 = ((mx1 + mx2) / 2.0 - pcx) * p_scr[5:6, :]
        g1 = ((my1 + my2) / 2.0 - pcy) * p_scr[6:7, :]
        g2 = jnp.log((mx2 - mx1) * p_scr[7:8, :]) / _VAR1
        g3 = jnp.log((my2 - my1) * p_scr[8:9, :]) / _VAR1

        def sl1(d):
            a = jnp.abs(d)
            return jnp.where(a < 1.0, 0.5 * d * d, a - 0.5)

        loc = loc_ref[r]
        lrow = (sl1(loc[0:1] - g0) + sl1(loc[1:2] - g1) +
                sl1(loc[2:3] - g2) + sl1(loc[3:4] - g3))
        ll = jnp.sum(jnp.where(pos, lrow, 0.0), axis=1, keepdims=True)

        cf = conf_ref[r]
        cmax = jnp.max(cf, axis=0, keepdims=True)
        e = jnp.exp(cf - cmax)
        lse = jnp.log(_dotf(ones_c, e)) + cmax
        gat = _dotf(ones_c, jnp.where(conf_t == i_cls, cf, 0.0))
        ce = lse - gat

        posf = pos.astype(jnp.float32)
        npos = jnp.sum(posf, axis=1, keepdims=True)
        pce = jnp.sum(jnp.where(pos, ce, 0.0), axis=1, keepdims=True)

        store(2 * g + r, jnp.where(pos, 0.0, ce), npos, ll, pce)

    one_image(0)
    one_image(1)


def _half_a_kernel(tgt_ref, tgtT_ref, pri_ref, conf_ref, loc_ref,
                   v_out, np_out, ll_out, pc_out, p_scr):
    g = pl.program_id(0)

    @pl.when(g == 0)
    def _prep():
        _prep_priors(pri_ref, p_scr)

    def store(row, v, npos, ll, pce):
        v_out[pl.ds(row, 1), :] = v
        np_out[pl.ds(row, 1), :] = npos
        ll_out[pl.ds(row, 1), :] = ll
        pc_out[pl.ds(row, 1), :] = pce

    _two_images(g, tgt_ref, tgtT_ref, pri_ref, conf_ref, loc_ref, p_scr,
                store)


def _half_b_kernel(tgt_ref, tgtT_ref, pri_ref, conf_ref, loc_ref,
                   v1_ref, np1_ref, ll1_ref, pc1_ref,
                   out_l_ref, out_c_ref,
                   v_scr, np_scr, ll_scr, pc_scr, p_scr):
    g = pl.program_id(0)
    G = pl.num_programs(0)
    P = pri_ref.shape[1]
    H = v1_ref.shape[0]
    B = 2 * H

    @pl.when(g == 0)
    def _prep():
        _prep_priors(pri_ref, p_scr)
        v_scr[0:H, :] = v1_ref[:, :]
        np_scr[0:H, :] = np1_ref[:, :]
        ll_scr[0:H, :] = ll1_ref[:, :]
        pc_scr[0:H, :] = pc1_ref[:, :]

    def store(row, v, npos, ll, pce):
        v_scr[pl.ds(H + row, 1), :] = v
        np_scr[pl.ds(H + row, 1), :] = npos
        ll_scr[pl.ds(H + row, 1), :] = ll
        pc_scr[pl.ds(H + row, 1), :] = pce

    _two_images(g, tgt_ref, tgtT_ref, pri_ref, conf_ref, loc_ref, p_scr,
                store)

    out_l_ref[:, :] = jnp.zeros((1, 1), jnp.float32)
    out_c_ref[:, :] = jnp.zeros((1, 1), jnp.float32)

    @pl.when(g == G - 1)
    def _finalize():
        v = v_scr[:, :]
        vb = jax.lax.bitcast_convert_type(v, jnp.int32)
        npf = np_scr[:, :]
        kf = jnp.minimum(_RATIO * npf, float(P - 1))

        def body(_, lh):
            lo, hi = lh
            mid = lo + (hi - lo + 1) // 2
            cnt = jnp.sum((vb >= mid).astype(jnp.float32), axis=1,
                          keepdims=True)
            ok = cnt >= kf
            return jnp.where(ok, mid, lo), jnp.where(ok, hi, mid - 1)

        lo0 = jnp.zeros((B, 1), jnp.int32)
        hi0 = jnp.full((B, 1), 0x7F7FFFFF, jnp.int32)
        lo, _ = jax.lax.fori_loop(0, 31, body, (lo0, hi0))
        tf = jax.lax.bitcast_convert_type(lo, jnp.float32)
        gt = vb > lo
        sgt = jnp.sum(jnp.where(gt, v, 0.0), axis=1, keepdims=True)
        cgt = jnp.sum(gt.astype(jnp.float32), axis=1, keepdims=True)
        top = sgt + (kf - cgt) * tf

        loss_c = jnp.sum(pc_scr[:, :] + top, axis=0, keepdims=True)
        loss_l = jnp.sum(ll_scr[:, :], axis=0, keepdims=True)
        nn = jnp.sum(np_scr[:, :], axis=0, keepdims=True)
        out_l_ref[:, :] = loss_l / nn
        out_c_ref[:, :] = loss_c / nn


def kernel(loc_data, conf_data, priors, targets):
    B, P, C = conf_data.shape
    n = targets.shape[1]
    H = B // 2
    pri_t = priors.T
    conf_a = conf_data[:H].transpose(0, 2, 1)
    conf_b = conf_data[H:].transpose(0, 2, 1)
    loc_cp = loc_data.transpose(0, 2, 1)
    tgt_t = targets.transpose(0, 2, 1)

    common_in = [
        pl.BlockSpec((2, n, 5), lambda b: (b, 0, 0)),
        pl.BlockSpec((2, 5, n), lambda b: (b, 0, 0)),
        pl.BlockSpec((4, P), lambda b: (0, 0)),
        pl.BlockSpec((2, C, P), lambda b: (b, 0, 0)),
        pl.BlockSpec((2, 4, P), lambda b: (b, 0, 0)),
    ]
    row_specs = [
        pl.BlockSpec((H, P), lambda b: (0, 0)),
        pl.BlockSpec((H, 1), lambda b: (0, 0)),
        pl.BlockSpec((H, 1), lambda b: (0, 0)),
        pl.BlockSpec((H, 1), lambda b: (0, 0)),
    ]
    row_shapes = [
        jax.ShapeDtypeStruct((H, P), jnp.float32),
        jax.ShapeDtypeStruct((H, 1), jnp.float32),
        jax.ShapeDtypeStruct((H, 1), jnp.float32),
        jax.ShapeDtypeStruct((H, 1), jnp.float32),
    ]
    params = pltpu.CompilerParams(dimension_semantics=("arbitrary",))

    v1, np1, ll1, pc1 = pl.pallas_call(
        _half_a_kernel,
        grid=(H // 2,),
        in_specs=common_in,
        out_specs=row_specs,
        out_shape=row_shapes,
        scratch_shapes=[pltpu.VMEM((16, P), jnp.float32)],
        compiler_params=params,
    )(targets[:H], tgt_t[:H], pri_t, conf_a, loc_cp[:H])

    out_l, out_c = pl.pallas_call(
        _half_b_kernel,
        grid=(H // 2,),
        in_specs=common_in + row_specs,
        out_specs=[
            pl.BlockSpec((1, 1), lambda b: (0, 0)),
            pl.BlockSpec((1, 1), lambda b: (0, 0)),
        ],
        out_shape=[
            jax.ShapeDtypeStruct((1, 1), jnp.float32),
            jax.ShapeDtypeStruct((1, 1), jnp.float32),
        ],
        scratch_shapes=[
            pltpu.VMEM((B, P), jnp.float32),
            pltpu.VMEM((B, 1), jnp.float32),
            pltpu.VMEM((B, 1), jnp.float32),
            pltpu.VMEM((B, 1), jnp.float32),
            pltpu.VMEM((16, P), jnp.float32),
        ],
        compiler_params=params,
    )(targets[H:], tgt_t[H:], pri_t, conf_b, loc_cp[H:],
      v1, np1, ll1, pc1)
    return out_l[0, 0], out_c[0, 0]
